# baseline (speedup 1.0000x reference)
import functools

import jax
import jax.numpy as jnp
from jax import lax
from jax.experimental import pallas as pl
from jax.experimental.pallas import tpu as pltpu
from jax.experimental.pallas import tpu_sc as plsc

_NC = 2
_NS = 16
_NW = _NC * _NS
_LANES = 16
_BPW = 128


def _make_sc_kernel(bsz, seq, d):
    assert d == 4 * _LANES
    assert bsz == _NW * _BPW
    assert seq % 2 == 0
    mesh = plsc.VectorSubcoreMesh(
        core_axis_name="c", subcore_axis_name="s",
        num_cores=_NC, num_subcores=_NS)

    @functools.partial(
        pl.kernel,
        out_type=jax.ShapeDtypeStruct((seq, d, bsz), jnp.float32),
        mesh=mesh,
        compiler_params=pltpu.CompilerParams(
            needs_layout_passes=False, use_tc_tiling_on_sc=True),
        scratch_types=[
            pltpu.VMEM((_BPW * seq,), jnp.int32),
            pltpu.VMEM((seq, _BPW), jnp.int32),
            pltpu.VMEM((_BPW, 2 * d), jnp.float32),
            pltpu.VMEM((_BPW, 2 * d), jnp.float32),
            pltpu.VMEM((d, _BPW), jnp.float32),
            pltpu.VMEM((d, _BPW), jnp.float32),
            pltpu.VMEM((d,), jnp.float32),
            pltpu.VMEM((d,), jnp.float32),
            pltpu.SemaphoreType.DMA,
            pltpu.SemaphoreType.DMA,
            pltpu.SemaphoreType.DMA,
            pltpu.SemaphoreType.DMA,
        ],
    )
    def sc_kernel(x_hbm, table_hbm, gamma_hbm, beta_hbm, out_hbm,
                  xs_v, xt_v, rows0_v, rows1_v, t0_v, t1_v,
                  g_v, b_v, sg0, sg1, sw0, sw1):
        wid = lax.axis_index("s") * _NC + lax.axis_index("c")
        rows = (rows0_v, rows1_v)
        tbuf = (t0_v, t1_v)
        sg = (sg0, sg1)
        sw = (sw0, sw1)
        b0 = wid * _BPW

        pltpu.sync_copy(x_hbm.at[pl.ds(b0 * seq, _BPW * seq)], xs_v)
        pltpu.sync_copy(gamma_hbm, g_v)
        pltpu.sync_copy(beta_hbm, b_v)

        lane = lax.iota(jnp.int32, _LANES)
        lane_seq = lane * seq

        @pl.loop(0, seq)
        def _tr(l):
            for j in range(_BPW // _LANES):
                g = plsc.load_gather(xs_v, [lane_seq + (j * _LANES * seq + l)])
                xt_v[l, pl.ds(j * _LANES, _LANES)] = g

        def gather_descr(l, bi):
            return pltpu.make_async_copy(
                table_hbm.at[xt_v.at[l]], rows[bi], sg[bi])

        def wb_descr(l, bi):
            return pltpu.make_async_copy(
                tbuf[bi], out_hbm.at[l, :, pl.ds(b0, _BPW)], sw[bi])

        def compute(bi):
            rv = rows[bi]
            tv = tbuf[bi]
            gv = [g_v[pl.ds(k * _LANES, _LANES)] for k in range(4)]
            bv = [b_v[pl.ds(k * _LANES, _LANES)] for k in range(4)]

            @pl.loop(0, _BPW, unroll=4)
            def _row(r):
                v = [rv[r, pl.ds(k * _LANES, _LANES)] for k in range(4)]
                s = (v[0] + v[1]) + (v[2] + v[3])
                q = (v[0] * v[0] + v[1] * v[1]) + (v[2] * v[2] + v[3] * v[3])
                mean = jnp.sum(s) * jnp.float32(1.0 / 64.0)
                ex2 = jnp.sum(q) * jnp.float32(1.0 / 64.0)
                xe = ex2 - mean * mean + jnp.float32(1e-5)
                i = lax.bitcast_convert_type(xe, jnp.int32)
                i = jnp.int32(0x5F3759DF) - lax.shift_right_logical(i, 1)
                y = lax.bitcast_convert_type(i, jnp.float32)
                nh = xe * jnp.float32(-0.5)
                for _ in range(3):
                    y = y * (jnp.float32(1.5) + nh * y * y)
                rs = jnp.full((_LANES,), y, jnp.float32)
                tm = jnp.full((_LANES,), mean * y, jnp.float32)
                ridx = jnp.full((_LANES,), r, jnp.int32)
                for k in range(4):
                    a = rs * gv[k]
                    cc2 = bv[k] - tm * gv[k]
                    plsc.store_scatter(
                        tv, [lane + (k * _LANES), ridx], v[k] * a + cc2)

        gather_descr(0, 0).start()

        @pl.loop(0, seq, step=2)
        def _chunks(c):
            for bi in range(2):
                l = c + bi

                @pl.when(l + 1 < seq)
                def _fire_next():
                    gather_descr(l + 1, bi ^ 1).start()

                gather_descr(l, bi).wait()

                @pl.when(l >= 2)
                def _wb_done():
                    wb_descr(l - 2, bi).wait()
                compute(bi)
                wb_descr(l, bi).start()

        wb_descr(seq - 2, 0).wait()
        wb_descr(seq - 1, 1).wait()

    return sc_kernel


def kernel(x, table, gamma, beta):
    bsz, seq = x.shape
    d = table.shape[1]
    tpad = jnp.pad(table, ((0, 0), (0, d)))
    out5 = _make_sc_kernel(bsz, seq, d)(
        x.reshape(-1).astype(jnp.int32), tpad, gamma, beta)
    return jnp.transpose(out5, (2, 0, 1))


# speedup vs baseline: 1.7962x; 1.7962x over previous
import functools

import jax
import jax.numpy as jnp
from jax import lax
from jax.experimental import pallas as pl
from jax.experimental.pallas import tpu as pltpu
from jax.experimental.pallas import tpu_sc as plsc

_NC = 2
_NS = 16
_NW = _NC * _NS
_LANES = 16

_G0 = 128


def _make_sc_kernel(bsz, seq, d):
    assert d == 4 * _LANES
    per_w = bsz // _NW
    assert per_w * _NW == bsz and per_w % 2 == 0
    assert seq > _G0 and seq - _G0 <= 128
    mesh = plsc.VectorSubcoreMesh(
        core_axis_name="c", subcore_axis_name="s",
        num_cores=_NC, num_subcores=_NS)

    @functools.partial(
        pl.kernel,
        out_type=jax.ShapeDtypeStruct((bsz, seq, d), jnp.float32),
        mesh=mesh,
        compiler_params=pltpu.CompilerParams(
            needs_layout_passes=False, use_tc_tiling_on_sc=False),
        scratch_types=[
            pltpu.VMEM((per_w, seq), jnp.int32),
            pltpu.VMEM((seq, d), jnp.float32),
            pltpu.VMEM((seq, d), jnp.float32),
            pltpu.VMEM((d,), jnp.float32),
            pltpu.VMEM((d,), jnp.float32),
            pltpu.SemaphoreType.DMA,
            pltpu.SemaphoreType.DMA,
            pltpu.SemaphoreType.DMA,
            pltpu.SemaphoreType.DMA,
        ],
    )
    def sc_kernel(x_hbm, table_hbm, gamma_hbm, beta_hbm, out_hbm,
                  idx_v, rows0_v, rows1_v, g_v, b_v, sg0, sg1, sw0, sw1):
        wid = lax.axis_index("s") * _NC + lax.axis_index("c")
        rows = (rows0_v, rows1_v)
        sg = (sg0, sg1)
        sw = (sw0, sw1)
        base_b = wid * per_w

        pltpu.sync_copy(x_hbm.at[pl.ds(base_b, per_w)], idx_v)
        pltpu.sync_copy(gamma_hbm, g_v)
        pltpu.sync_copy(beta_hbm, b_v)

        def gather_descrs(cc, bi):
            return [
                pltpu.make_async_copy(
                    table_hbm.at[idx_v.at[cc, pl.ds(0, _G0)]],
                    rows[bi].at[pl.ds(0, _G0)], sg[bi]),
                pltpu.make_async_copy(
                    table_hbm.at[idx_v.at[cc, pl.ds(_G0, seq - _G0)]],
                    rows[bi].at[pl.ds(_G0, seq - _G0)], sg[bi]),
            ]

        def wb_descr(cc, bi):
            return pltpu.make_async_copy(
                rows[bi], out_hbm.at[base_b + cc], sw[bi])

        def compute(bi):
            rv = rows[bi]
            gv = [g_v[pl.ds(k * _LANES, _LANES)] for k in range(4)]
            bv = [b_v[pl.ds(k * _LANES, _LANES)] for k in range(4)]

            @pl.loop(0, seq, unroll=8)
            def _row(r):
                v = [rv[r, pl.ds(k * _LANES, _LANES)] for k in range(4)]
                s = (v[0] + v[1]) + (v[2] + v[3])
                q = (v[0] * v[0] + v[1] * v[1]) + (v[2] * v[2] + v[3] * v[3])
                mean = jnp.sum(s) * jnp.float32(1.0 / 64.0)
                ex2 = jnp.sum(q) * jnp.float32(1.0 / 64.0)
                xe = ex2 - mean * mean + jnp.float32(1e-5)
                i = lax.bitcast_convert_type(xe, jnp.int32)
                i = jnp.int32(0x5F3759DF) - lax.shift_right_logical(i, 1)
                y = lax.bitcast_convert_type(i, jnp.float32)
                nh = xe * jnp.float32(-0.5)
                for _ in range(3):
                    y = y * (jnp.float32(1.5) + nh * y * y)
                rs = jnp.full((_LANES,), y, jnp.float32)
                tm = jnp.full((_LANES,), mean * y, jnp.float32)
                for k in range(4):
                    a = rs * gv[k]
                    cc2 = bv[k] - tm * gv[k]
                    rv[r, pl.ds(k * _LANES, _LANES)] = v[k] * a + cc2

        for g in gather_descrs(0, 0):
            g.start()

        @pl.loop(0, per_w, step=2)
        def _chunks(c):
            for bi in range(2):
                cc = c + bi

                @pl.when(cc + 1 < per_w)
                def _fire_next():
                    @pl.when(cc >= 1)
                    def _wb_done():
                        wb_descr(cc - 1, bi ^ 1).wait()
                    for g in gather_descrs(cc + 1, bi ^ 1):
                        g.start()

                for g in gather_descrs(cc, bi):
                    g.wait()
                compute(bi)
                wb_descr(cc, bi).start()

        wb_descr(per_w - 2, 0).wait()
        wb_descr(per_w - 1, 1).wait()

    return sc_kernel


def kernel(x, table, gamma, beta):
    bsz, seq = x.shape
    d = table.shape[1]
    return _make_sc_kernel(bsz, seq, d)(
        x.astype(jnp.int32), table, gamma, beta)
